# Initial kernel scaffold; baseline (speedup 1.0000x reference)
import jax
import jax.numpy as jnp
from jax.experimental import pallas as pl


def kernel(species, positions, senders, receivers, W0, b0, Wv, W1, b1, Wout):
    raise NotImplementedError("write your pallas kernel here")


# speedup vs baseline: 10.4784x; 10.4784x over previous
import functools
import math

import jax
import jax.numpy as jnp
from jax import lax
from jax.experimental import pallas as pl
from jax.experimental.pallas import tpu as pltpu
from jax.experimental.pallas import tpu_sc as plsc

N = 50000
E = 800000
HID = 64
VHID = 16
NB = 8
RC = 2.0
AVG_NEIGH = 3.0

NC = 2
NS = 16
NW = NC * NS

_SC_PARAMS = pltpu.CompilerParams(use_tc_tiling_on_sc=False)


_P1_CHUNK = 5000
_P1_PER_W = E // NW


def _sc_gather_body(table_hbm, senders_hbm, receivers_hbm, ps_hbm, pr_hbm,
                    idx_v, rows_v, sem):
    c = lax.axis_index("c")
    s = lax.axis_index("s")
    w = s * NC + c
    base = w * _P1_PER_W
    for src_hbm, dst_hbm in ((senders_hbm, ps_hbm), (receivers_hbm, pr_hbm)):
        for j in range(_P1_PER_W // _P1_CHUNK):
            off = base + j * _P1_CHUNK
            pltpu.sync_copy(src_hbm.at[pl.ds(off, _P1_CHUNK)], idx_v)
            pltpu.async_copy(table_hbm.at[idx_v], rows_v, sem).wait()
            pltpu.sync_copy(rows_v, dst_hbm.at[pl.ds(off, _P1_CHUNK)])


def _sc_gather(table, senders, receivers):
    mesh = plsc.VectorSubcoreMesh(core_axis_name="c", subcore_axis_name="s")
    kfn = pl.kernel(
        _sc_gather_body,
        mesh=mesh,
        out_type=[
            jax.ShapeDtypeStruct((E, 16), jnp.float32),
            jax.ShapeDtypeStruct((E, 16), jnp.float32),
        ],
        scratch_types=[
            pltpu.VMEM((_P1_CHUNK,), jnp.int32),
            pltpu.VMEM((_P1_CHUNK, 16), jnp.float32),
            pltpu.SemaphoreType.DMA,
        ],
        compiler_params=_SC_PARAMS,
    )
    return kfn(table, senders, receivers)


_BE = 4000


def _edge_geom(psb, prb):
    ps = psb[:, 0:3]
    pr = prb[:, 0:3]
    vec = pr - ps
    r2 = jnp.sum(vec * vec, axis=1, keepdims=True) + 1e-12
    r = jnp.sqrt(r2)
    rinv = 1.0 / r
    vhat = vec * rinv
    d = r * (1.0 / RC)
    p = 6.0
    env = (1.0
           - ((p + 1.0) * (p + 2.0) / 2.0) * d ** 6
           + p * (p + 2.0) * d ** 7
           - (p * (p + 1.0) / 2.0) * d ** 8)
    env = jnp.where(d < 1.0, env, 0.0)
    n = lax.broadcasted_iota(jnp.int32, (1, NB), 1).astype(jnp.float32) + 1.0
    bessel = math.sqrt(2.0 / RC) * jnp.sin(n * jnp.pi * d) * rinv
    emb16 = jnp.concatenate(
        [bessel, psb[:, 3:5], prb[:, 3:5], jnp.zeros((psb.shape[0], 4), jnp.float32)],
        axis=1)
    return vhat, env, emb16


def _p2_body(ps_ref, pr_ref, w0_ref, b0_ref, wv_ref, v0_ref, v1_ref):
    psb = ps_ref[...]
    prb = pr_ref[...]
    vhat, env, emb16 = _edge_geom(psb, prb)
    z = jnp.dot(emb16, w0_ref[...], precision=lax.Precision.HIGHEST) + b0_ref[...]
    h = z * jax.nn.sigmoid(z)
    scale = env * (1.0 / math.sqrt(AVG_NEIGH))
    hwv = jnp.dot(h, wv_ref[...], precision=lax.Precision.HIGHEST) * scale
    lo = hwv[:, 0:8]
    hi = hwv[:, 8:16]
    v0_ref[...] = jnp.concatenate(
        [lo * vhat[:, 0:1], lo * vhat[:, 1:2], lo * vhat[:, 2:3]], axis=1)
    v1_ref[...] = jnp.concatenate(
        [hi * vhat[:, 0:1], hi * vhat[:, 1:2], hi * vhat[:, 2:3]], axis=1)


def _p2(ps, pr, w0p, b0, wv):
    grid = (E // _BE,)
    return pl.pallas_call(
        _p2_body,
        grid=grid,
        in_specs=[
            pl.BlockSpec((_BE, 16), lambda i: (i, 0)),
            pl.BlockSpec((_BE, 16), lambda i: (i, 0)),
            pl.BlockSpec((16, HID), lambda i: (0, 0)),
            pl.BlockSpec((1, HID), lambda i: (0, 0)),
            pl.BlockSpec((HID, VHID), lambda i: (0, 0)),
        ],
        out_specs=[
            pl.BlockSpec((_BE, 24), lambda i: (i, 0)),
            pl.BlockSpec((_BE, 24), lambda i: (i, 0)),
        ],
        out_shape=[
            jax.ShapeDtypeStruct((E, 24), jnp.float32),
            jax.ShapeDtypeStruct((E, 24), jnp.float32),
        ],
    )(ps, pr, w0p, b0, wv)


_P3_CHUNK = 1000
_P3_PER_T = E // NS


def _p3_body(senders_hbm, v0_hbm, v1_hbm, az_hbm,
             a0_hbm, a1_hbm, ag0_hbm, ag1_hbm,
             idx_v, buf_v, sem, a_sh):
    c = lax.axis_index("c")
    s = lax.axis_index("s")

    @pl.when(s == 0)
    def _init():
        pltpu.sync_copy(az_hbm, a_sh)

    plsc.subcore_barrier()

    def _stage_idx(j):
        off = s * _P3_PER_T + j * _P3_CHUNK
        pltpu.sync_copy(senders_hbm.at[pl.ds(off, _P3_CHUNK)], idx_v)
        return off

    for j in range(_P3_PER_T // _P3_CHUNK):
        off = _stage_idx(j)

        @pl.when(c == 0)
        def _ld0():
            pltpu.sync_copy(v0_hbm.at[pl.ds(off, _P3_CHUNK), :], buf_v)

        @pl.when(c == 1)
        def _ld1():
            pltpu.sync_copy(v1_hbm.at[pl.ds(off, _P3_CHUNK), :], buf_v)

        pltpu.sync_copy(buf_v, a_sh.at[idx_v], add=True)

    plsc.subcore_barrier()

    @pl.when(s == 0)
    def _writeback():
        @pl.when(c == 0)
        def _w0():
            pltpu.sync_copy(a_sh, a0_hbm)

        @pl.when(c == 1)
        def _w1():
            pltpu.sync_copy(a_sh, a1_hbm)

    plsc.subcore_barrier()

    for j in range(_P3_PER_T // _P3_CHUNK):
        off = _stage_idx(j)

        @pl.when(c == 0)
        def _g0():
            pltpu.async_copy(a0_hbm.at[idx_v], buf_v, sem).wait()
            pltpu.sync_copy(buf_v, ag0_hbm.at[pl.ds(off, _P3_CHUNK), :])

        @pl.when(c == 1)
        def _g1():
            pltpu.async_copy(a1_hbm.at[idx_v], buf_v, sem).wait()
            pltpu.sync_copy(buf_v, ag1_hbm.at[pl.ds(off, _P3_CHUNK), :])


def _p3(senders, v0, v1, az):
    mesh = plsc.VectorSubcoreMesh(core_axis_name="c", subcore_axis_name="s")
    kfn = pl.kernel(
        _p3_body,
        mesh=mesh,
        out_type=[
            jax.ShapeDtypeStruct((N, 24), jnp.float32),
            jax.ShapeDtypeStruct((N, 24), jnp.float32),
            jax.ShapeDtypeStruct((E, 24), jnp.float32),
            jax.ShapeDtypeStruct((E, 24), jnp.float32),
        ],
        scratch_types=[
            pltpu.VMEM((_P3_CHUNK,), jnp.int32),
            pltpu.VMEM((_P3_CHUNK, 24), jnp.float32),
            pltpu.SemaphoreType.DMA,
            pltpu.VMEM_SHARED((N, 24), jnp.float32),
        ],
        compiler_params=_SC_PARAMS,
    )
    return kfn(senders, v0, v1, az)


def _p4_body(ps_ref, pr_ref, ag0_ref, ag1_ref,
             w0_ref, b0_ref, w1_ref, b1_ref, wout_ref, out_ref):
    psb = ps_ref[...]
    prb = pr_ref[...]
    vhat, env, emb16 = _edge_geom(psb, prb)
    z = jnp.dot(emb16, w0_ref[...], precision=lax.Precision.HIGHEST) + b0_ref[...]
    h = (z * jax.nn.sigmoid(z)) * env
    ag0 = ag0_ref[...]
    ag1 = ag1_ref[...]
    s_lo = (ag0[:, 0:8] * vhat[:, 0:1]
            + ag0[:, 8:16] * vhat[:, 1:2]
            + ag0[:, 16:24] * vhat[:, 2:3])
    s_hi = (ag1[:, 0:8] * vhat[:, 0:1]
            + ag1[:, 8:16] * vhat[:, 1:2]
            + ag1[:, 16:24] * vhat[:, 2:3])
    cat = jnp.concatenate([h, s_lo, s_hi], axis=1)
    z2 = jnp.dot(cat, w1_ref[...], precision=lax.Precision.HIGHEST) + b1_ref[...]
    h2 = (z2 * jax.nn.sigmoid(z2)) * env
    out_ref[...] = jnp.dot(h2, wout_ref[...], precision=lax.Precision.HIGHEST)


def _p4(ps, pr, ag0, ag1, w0p, b0, w1, b1, wout):
    grid = (E // _BE,)
    return pl.pallas_call(
        _p4_body,
        grid=grid,
        in_specs=[
            pl.BlockSpec((_BE, 16), lambda i: (i, 0)),
            pl.BlockSpec((_BE, 16), lambda i: (i, 0)),
            pl.BlockSpec((_BE, 24), lambda i: (i, 0)),
            pl.BlockSpec((_BE, 24), lambda i: (i, 0)),
            pl.BlockSpec((16, HID), lambda i: (0, 0)),
            pl.BlockSpec((1, HID), lambda i: (0, 0)),
            pl.BlockSpec((HID + VHID, HID), lambda i: (0, 0)),
            pl.BlockSpec((1, HID), lambda i: (0, 0)),
            pl.BlockSpec((HID, 1), lambda i: (0, 0)),
        ],
        out_specs=pl.BlockSpec((_BE, 1), lambda i: (i, 0)),
        out_shape=jax.ShapeDtypeStruct((E, 1), jnp.float32),
    )(ps, pr, ag0, ag1, w0p, b0, w1, b1, wout)


def kernel(species, positions, senders, receivers, W0, b0, Wv, W1, b1, Wout):
    attrs = jax.nn.one_hot(species, 2, dtype=jnp.float32)
    table = jnp.concatenate(
        [positions, attrs, jnp.zeros((N, 11), jnp.float32)], axis=1)
    senders = senders.astype(jnp.int32)
    receivers = receivers.astype(jnp.int32)
    w0p = jnp.concatenate([W0, jnp.zeros((4, HID), jnp.float32)], axis=0)
    b0_2d = b0.reshape(1, HID)
    b1_2d = b1.reshape(1, HID)
    az = jnp.zeros((N, 24), jnp.float32)

    ps, pr = _sc_gather(table, senders, receivers)
    v0, v1 = _p2(ps, pr, w0p, b0_2d, Wv)
    _a0, _a1, ag0, ag1 = _p3(senders, v0, v1, az)
    return _p4(ps, pr, ag0, ag1, w0p, b0_2d, W1, b1_2d, Wout)


# speedup vs baseline: 26.4989x; 2.5289x over previous
import functools
import math

import jax
import jax.numpy as jnp
from jax import lax
from jax.experimental import pallas as pl
from jax.experimental.pallas import tpu as pltpu
from jax.experimental.pallas import tpu_sc as plsc

N = 50000
E = 800000
HID = 64
VHID = 16
NB = 8
RC = 2.0
AVG_NEIGH = 3.0

NC = 2
NS = 16
NW = NC * NS

_SC_PARAMS = pltpu.CompilerParams(use_tc_tiling_on_sc=False)


_P1_CHUNK = 5000
_P1_PER_W = E // NW


def _sc_gather_body(table_hbm, senders_hbm, receivers_hbm, ps_hbm, pr_hbm,
                    idx_v, rows_v, sem):
    c = lax.axis_index("c")
    s = lax.axis_index("s")
    w = s * NC + c
    base = w * _P1_PER_W
    for src_hbm, dst_hbm in ((senders_hbm, ps_hbm), (receivers_hbm, pr_hbm)):
        for j in range(_P1_PER_W // _P1_CHUNK):
            off = base + j * _P1_CHUNK
            pltpu.sync_copy(src_hbm.at[pl.ds(off, _P1_CHUNK)], idx_v)
            pltpu.async_copy(table_hbm.at[idx_v], rows_v, sem).wait()
            pltpu.sync_copy(rows_v, dst_hbm.at[pl.ds(off, _P1_CHUNK)])


def _sc_gather(table, senders, receivers):
    mesh = plsc.VectorSubcoreMesh(core_axis_name="c", subcore_axis_name="s")
    kfn = pl.kernel(
        _sc_gather_body,
        mesh=mesh,
        out_type=[
            jax.ShapeDtypeStruct((E, 16), jnp.float32),
            jax.ShapeDtypeStruct((E, 16), jnp.float32),
        ],
        scratch_types=[
            pltpu.VMEM((_P1_CHUNK,), jnp.int32),
            pltpu.VMEM((_P1_CHUNK, 16), jnp.float32),
            pltpu.SemaphoreType.DMA,
        ],
        compiler_params=_SC_PARAMS,
    )
    return kfn(table, senders, receivers)


_BE = 4000


def _edge_geom(psT, prT):
    B = psT.shape[1]
    vec = prT[0:3] - psT[0:3]
    r2 = jnp.sum(vec * vec, axis=0, keepdims=True) + 1e-12
    r = jnp.sqrt(r2)
    rinv = 1.0 / r
    vhat = vec * rinv
    d = r * (1.0 / RC)
    p = 6.0
    env = (1.0
           - ((p + 1.0) * (p + 2.0) / 2.0) * d ** 6
           + p * (p + 2.0) * d ** 7
           - (p * (p + 1.0) / 2.0) * d ** 8)
    env = jnp.where(d < 1.0, env, 0.0)
    x = jnp.pi * d
    s1 = jnp.sin(x)
    two_c = 2.0 * jnp.cos(x)
    sins = [s1, two_c * s1]
    for _ in range(NB - 2):
        sins.append(two_c * sins[-1] - sins[-2])
    bessel = jnp.concatenate(sins, axis=0) * (math.sqrt(2.0 / RC) * rinv)
    emb16 = jnp.concatenate(
        [bessel, psT[3:5], prT[3:5], jnp.zeros((4, B), jnp.float32)], axis=0)
    return vhat, env, emb16


def _p2_body(ps_ref, pr_ref, w0t_ref, b0_ref, wvt_ref, v0_ref, v1_ref):
    psT = ps_ref[...].T
    prT = pr_ref[...].T
    vhat, env, emb16 = _edge_geom(psT, prT)
    z = jnp.dot(w0t_ref[...], emb16, precision=lax.Precision.HIGHEST) + b0_ref[...]
    h = z * jax.nn.sigmoid(z)
    scale = env * (1.0 / math.sqrt(AVG_NEIGH))
    hwv = jnp.dot(wvt_ref[...], h, precision=lax.Precision.HIGHEST) * scale
    lo = hwv[0:8]
    hi = hwv[8:16]
    v0_ref[...] = jnp.concatenate(
        [lo * vhat[0:1], lo * vhat[1:2], lo * vhat[2:3]], axis=0).T
    v1_ref[...] = jnp.concatenate(
        [hi * vhat[0:1], hi * vhat[1:2], hi * vhat[2:3]], axis=0).T


def _p2(ps, pr, w0p, b0, wv):
    grid = (E // _BE,)
    return pl.pallas_call(
        _p2_body,
        grid=grid,
        in_specs=[
            pl.BlockSpec((_BE, 16), lambda i: (i, 0)),
            pl.BlockSpec((_BE, 16), lambda i: (i, 0)),
            pl.BlockSpec((HID, 16), lambda i: (0, 0)),
            pl.BlockSpec((HID, 1), lambda i: (0, 0)),
            pl.BlockSpec((VHID, HID), lambda i: (0, 0)),
        ],
        out_specs=[
            pl.BlockSpec((_BE, 24), lambda i: (i, 0)),
            pl.BlockSpec((_BE, 24), lambda i: (i, 0)),
        ],
        out_shape=[
            jax.ShapeDtypeStruct((E, 24), jnp.float32),
            jax.ShapeDtypeStruct((E, 24), jnp.float32),
        ],
    )(ps, pr, w0p, b0, wv)


_P3_CHUNK = 1000
_P3_PER_T = E // NS


def _p3_body(senders_hbm, v0_hbm, v1_hbm, az_hbm,
             a0_hbm, a1_hbm, ag0_hbm, ag1_hbm,
             idx_v, buf_v, sem, a_sh):
    c = lax.axis_index("c")
    s = lax.axis_index("s")

    @pl.when(s == 0)
    def _init():
        pltpu.sync_copy(az_hbm, a_sh)

    plsc.subcore_barrier()

    def _stage_idx(j):
        off = s * _P3_PER_T + j * _P3_CHUNK
        pltpu.sync_copy(senders_hbm.at[pl.ds(off, _P3_CHUNK)], idx_v)
        return off

    for j in range(_P3_PER_T // _P3_CHUNK):
        off = _stage_idx(j)

        @pl.when(c == 0)
        def _ld0():
            pltpu.sync_copy(v0_hbm.at[pl.ds(off, _P3_CHUNK), :], buf_v)

        @pl.when(c == 1)
        def _ld1():
            pltpu.sync_copy(v1_hbm.at[pl.ds(off, _P3_CHUNK), :], buf_v)

        pltpu.sync_copy(buf_v, a_sh.at[idx_v], add=True)

    plsc.subcore_barrier()

    @pl.when(s == 0)
    def _writeback():
        @pl.when(c == 0)
        def _w0():
            pltpu.sync_copy(a_sh, a0_hbm)

        @pl.when(c == 1)
        def _w1():
            pltpu.sync_copy(a_sh, a1_hbm)

    plsc.subcore_barrier()

    for j in range(_P3_PER_T // _P3_CHUNK):
        off = _stage_idx(j)

        @pl.when(c == 0)
        def _g0():
            pltpu.async_copy(a0_hbm.at[idx_v], buf_v, sem).wait()
            pltpu.sync_copy(buf_v, ag0_hbm.at[pl.ds(off, _P3_CHUNK), :])

        @pl.when(c == 1)
        def _g1():
            pltpu.async_copy(a1_hbm.at[idx_v], buf_v, sem).wait()
            pltpu.sync_copy(buf_v, ag1_hbm.at[pl.ds(off, _P3_CHUNK), :])


def _p3(senders, v0, v1, az):
    mesh = plsc.VectorSubcoreMesh(core_axis_name="c", subcore_axis_name="s")
    kfn = pl.kernel(
        _p3_body,
        mesh=mesh,
        out_type=[
            jax.ShapeDtypeStruct((N, 24), jnp.float32),
            jax.ShapeDtypeStruct((N, 24), jnp.float32),
            jax.ShapeDtypeStruct((E, 24), jnp.float32),
            jax.ShapeDtypeStruct((E, 24), jnp.float32),
        ],
        scratch_types=[
            pltpu.VMEM((_P3_CHUNK,), jnp.int32),
            pltpu.VMEM((_P3_CHUNK, 24), jnp.float32),
            pltpu.SemaphoreType.DMA,
            pltpu.VMEM_SHARED((N, 24), jnp.float32),
        ],
        compiler_params=_SC_PARAMS,
    )
    return kfn(senders, v0, v1, az)


def _p4_body(ps_ref, pr_ref, ag0_ref, ag1_ref,
             w0t_ref, b0_ref, w1t_ref, b1_ref, woutt_ref, out_ref):
    psT = ps_ref[...].T
    prT = pr_ref[...].T
    vhat, env, emb16 = _edge_geom(psT, prT)
    z = jnp.dot(w0t_ref[...], emb16, precision=lax.Precision.HIGHEST) + b0_ref[...]
    h = (z * jax.nn.sigmoid(z)) * env
    ag0 = ag0_ref[...].T
    ag1 = ag1_ref[...].T
    s_lo = (ag0[0:8] * vhat[0:1]
            + ag0[8:16] * vhat[1:2]
            + ag0[16:24] * vhat[2:3])
    s_hi = (ag1[0:8] * vhat[0:1]
            + ag1[8:16] * vhat[1:2]
            + ag1[16:24] * vhat[2:3])
    cat = jnp.concatenate([h, s_lo, s_hi], axis=0)
    z2 = jnp.dot(w1t_ref[...], cat, precision=lax.Precision.HIGHEST) + b1_ref[...]
    h2 = (z2 * jax.nn.sigmoid(z2)) * env
    out_ref[...] = jnp.dot(
        woutt_ref[...], h2, precision=lax.Precision.HIGHEST).T


def _p4(ps, pr, ag0, ag1, w0p, b0, w1, b1, wout):
    grid = (E // _BE,)
    return pl.pallas_call(
        _p4_body,
        grid=grid,
        in_specs=[
            pl.BlockSpec((_BE, 16), lambda i: (i, 0)),
            pl.BlockSpec((_BE, 16), lambda i: (i, 0)),
            pl.BlockSpec((_BE, 24), lambda i: (i, 0)),
            pl.BlockSpec((_BE, 24), lambda i: (i, 0)),
            pl.BlockSpec((HID, 16), lambda i: (0, 0)),
            pl.BlockSpec((HID, 1), lambda i: (0, 0)),
            pl.BlockSpec((HID, HID + VHID), lambda i: (0, 0)),
            pl.BlockSpec((HID, 1), lambda i: (0, 0)),
            pl.BlockSpec((1, HID), lambda i: (0, 0)),
        ],
        out_specs=pl.BlockSpec((_BE, 1), lambda i: (i, 0)),
        out_shape=jax.ShapeDtypeStruct((E, 1), jnp.float32),
    )(ps, pr, ag0, ag1, w0p, b0, w1, b1, wout)


def kernel(species, positions, senders, receivers, W0, b0, Wv, W1, b1, Wout):
    attrs = jax.nn.one_hot(species, 2, dtype=jnp.float32)
    table = jnp.concatenate(
        [positions, attrs, jnp.zeros((N, 11), jnp.float32)], axis=1)
    senders = senders.astype(jnp.int32)
    receivers = receivers.astype(jnp.int32)
    w0t = jnp.concatenate([W0, jnp.zeros((4, HID), jnp.float32)], axis=0).T
    wvt = Wv.T
    w1t = W1.T
    woutt = Wout.T
    b0c = b0.reshape(HID, 1)
    b1c = b1.reshape(HID, 1)
    az = jnp.zeros((N, 24), jnp.float32)

    ps, pr = _sc_gather(table, senders, receivers)
    v0, v1 = _p2(ps, pr, w0t, b0c, wvt)
    _a0, _a1, ag0, ag1 = _p3(senders, v0, v1, az)
    return _p4(ps, pr, ag0, ag1, w0t, b0c, w1t, b1c, woutt)


# speedup vs baseline: 46.7677x; 1.7649x over previous
import functools
import math

import jax
import jax.numpy as jnp
from jax import lax
from jax.experimental import pallas as pl
from jax.experimental.pallas import tpu as pltpu
from jax.experimental.pallas import tpu_sc as plsc

N = 50000
E = 800000
HID = 64
VHID = 16
NB = 8
RC = 2.0
AVG_NEIGH = 3.0

NC = 2
NS = 16
NW = NC * NS

_SC_PARAMS = pltpu.CompilerParams(use_tc_tiling_on_sc=False)

_EPK = E * 16 // 128


_P1_CHUNK = 1000
_P1_PER_W = E // NW


def _repack_rows(rows_v, pk_v, n_rows):
    @pl.loop(0, n_rows // 8)
    def _row(r):
        for k in range(8):
            pk_v[r, pl.ds(k * 16, 16)] = rows_v[r * 8 + k, :]


def _unpack_rows(pk_v, rows_v, n_rows):
    @pl.loop(0, n_rows // 8)
    def _row(r):
        for k in range(8):
            rows_v[r * 8 + k, :] = pk_v[r, pl.ds(k * 16, 16)]


def _sc_gather_body(table_hbm, senders_hbm, receivers_hbm, ps_hbm, pr_hbm,
                    idx_v, rows_v, pk_v, sem):
    c = lax.axis_index("c")
    s = lax.axis_index("s")
    w = s * NC + c
    base = w * _P1_PER_W
    for src_hbm, dst_hbm in ((senders_hbm, ps_hbm), (receivers_hbm, pr_hbm)):
        @pl.loop(0, _P1_PER_W // _P1_CHUNK)
        def _chunk(j):
            off = base + j * _P1_CHUNK
            pltpu.sync_copy(src_hbm.at[pl.ds(off, _P1_CHUNK)], idx_v)
            pltpu.async_copy(table_hbm.at[idx_v], rows_v, sem).wait()
            _repack_rows(rows_v, pk_v, _P1_CHUNK)
            pltpu.sync_copy(
                pk_v, dst_hbm.at[pl.ds(off // 8, _P1_CHUNK // 8)])


def _sc_gather(table, senders, receivers):
    mesh = plsc.VectorSubcoreMesh(core_axis_name="c", subcore_axis_name="s")
    kfn = pl.kernel(
        _sc_gather_body,
        mesh=mesh,
        out_type=[
            jax.ShapeDtypeStruct((_EPK, 128), jnp.float32),
            jax.ShapeDtypeStruct((_EPK, 128), jnp.float32),
        ],
        scratch_types=[
            pltpu.VMEM((_P1_CHUNK,), jnp.int32),
            pltpu.VMEM((_P1_CHUNK, 16), jnp.float32),
            pltpu.VMEM((_P1_CHUNK // 8, 128), jnp.float32),
            pltpu.SemaphoreType.DMA,
        ],
        compiler_params=_SC_PARAMS,
    )
    return kfn(table, senders, receivers)


_BE = 6400


def _unpack16(pk):
    t = pk.T
    parts = [t[16 * j:16 * j + 16, :] for j in range(8)]
    return jnp.concatenate(parts, axis=1)


def _pack16(x):
    r = x.shape[1] // 8
    parts = [x[:, j * r:(j + 1) * r] for j in range(8)]
    return jnp.concatenate(parts, axis=0).T


def _pack_out(x):
    r = x.shape[1] // 8
    parts = [x[:, j * r:(j + 1) * r] for j in range(8)]
    return jnp.concatenate(parts, axis=0).T


def _edge_geom(psT, prT):
    B = psT.shape[1]
    vec = prT[0:3] - psT[0:3]
    r2 = jnp.sum(vec * vec, axis=0, keepdims=True) + 1e-12
    r = jnp.sqrt(r2)
    rinv = 1.0 / r
    vhat = vec * rinv
    d = r * (1.0 / RC)
    p = 6.0
    env = (1.0
           - ((p + 1.0) * (p + 2.0) / 2.0) * d ** 6
           + p * (p + 2.0) * d ** 7
           - (p * (p + 1.0) / 2.0) * d ** 8)
    env = jnp.where(d < 1.0, env, 0.0)
    x = jnp.pi * d
    s1 = jnp.sin(x)
    two_c = 2.0 * jnp.cos(x)
    sins = [s1, two_c * s1]
    for _ in range(NB - 2):
        sins.append(two_c * sins[-1] - sins[-2])
    bessel = jnp.concatenate(sins, axis=0) * (math.sqrt(2.0 / RC) * rinv)
    emb16 = jnp.concatenate(
        [bessel, psT[3:5], prT[3:5], jnp.zeros((4, B), jnp.float32)], axis=0)
    return vhat, env, emb16


def _p2_body(ps_ref, pr_ref, w0t_ref, b0_ref, wvt_ref,
             v0a_ref, v0b_ref, v1a_ref, v1b_ref):
    psT = _unpack16(ps_ref[...])
    prT = _unpack16(pr_ref[...])
    vhat, env, emb16 = _edge_geom(psT, prT)
    z = jnp.dot(w0t_ref[...], emb16, precision=lax.Precision.HIGHEST) + b0_ref[...]
    h = z * jax.nn.sigmoid(z)
    scale = env * (1.0 / math.sqrt(AVG_NEIGH))
    hwv = jnp.dot(wvt_ref[...], h, precision=lax.Precision.HIGHEST) * scale
    lo = hwv[0:8]
    hi = hwv[8:16]
    zpad = jnp.zeros((8, hwv.shape[1]), jnp.float32)
    v0a_ref[...] = _pack16(
        jnp.concatenate([lo * vhat[0:1], lo * vhat[1:2]], axis=0))
    v0b_ref[...] = _pack16(jnp.concatenate([lo * vhat[2:3], zpad], axis=0))
    v1a_ref[...] = _pack16(
        jnp.concatenate([hi * vhat[0:1], hi * vhat[1:2]], axis=0))
    v1b_ref[...] = _pack16(jnp.concatenate([hi * vhat[2:3], zpad], axis=0))


def _p2(ps, pr, w0t, b0c, wvt):
    grid = (E // _BE,)
    vspec = pl.BlockSpec((_BE // 8, 128), lambda i: (i, 0))
    vshape = jax.ShapeDtypeStruct((_EPK, 128), jnp.float32)
    return pl.pallas_call(
        _p2_body,
        grid=grid,
        in_specs=[
            vspec,
            vspec,
            pl.BlockSpec((HID, 16), lambda i: (0, 0)),
            pl.BlockSpec((HID, 1), lambda i: (0, 0)),
            pl.BlockSpec((VHID, HID), lambda i: (0, 0)),
        ],
        out_specs=[vspec, vspec, vspec, vspec],
        out_shape=[vshape, vshape, vshape, vshape],
    )(ps, pr, w0t, b0c, wvt)


_P3_CHUNK = 400
_P3_PER_T = E // NS


def _p3_body(senders_hbm, v0a_hbm, v0b_hbm, v1a_hbm, v1b_hbm, az_hbm,
             a0a_hbm, a0b_hbm, a1a_hbm, a1b_hbm,
             ag0a_hbm, ag0b_hbm, ag1a_hbm, ag1b_hbm,
             idx_v, rows_v, pk_v, sem, asha, ashb):
    c = lax.axis_index("c")
    s = lax.axis_index("s")

    @pl.when(s == 0)
    def _init():
        pltpu.sync_copy(az_hbm, asha)
        pltpu.sync_copy(az_hbm, ashb)

    plsc.subcore_barrier()

    def _stage_idx(j):
        off = s * _P3_PER_T + j * _P3_CHUNK
        pltpu.sync_copy(senders_hbm.at[pl.ds(off, _P3_CHUNK)], idx_v)
        return off

    @pl.loop(0, _P3_PER_T // _P3_CHUNK)
    def _scatter_chunk(j):
        off = _stage_idx(j)
        poff = off // 8
        for v0_hbm, v1_hbm, a_sh in ((v0a_hbm, v1a_hbm, asha),
                                     (v0b_hbm, v1b_hbm, ashb)):
            @pl.when(c == 0)
            def _ld0():
                pltpu.sync_copy(v0_hbm.at[pl.ds(poff, _P3_CHUNK // 8)], pk_v)

            @pl.when(c == 1)
            def _ld1():
                pltpu.sync_copy(v1_hbm.at[pl.ds(poff, _P3_CHUNK // 8)], pk_v)

            _unpack_rows(pk_v, rows_v, _P3_CHUNK)
            pltpu.sync_copy(rows_v, a_sh.at[idx_v], add=True)

    plsc.subcore_barrier()

    @pl.when(s == 0)
    def _writeback():
        @pl.when(c == 0)
        def _w0():
            pltpu.sync_copy(asha, a0a_hbm)
            pltpu.sync_copy(ashb, a0b_hbm)

        @pl.when(c == 1)
        def _w1():
            pltpu.sync_copy(asha, a1a_hbm)
            pltpu.sync_copy(ashb, a1b_hbm)

    plsc.subcore_barrier()

    @pl.loop(0, _P3_PER_T // _P3_CHUNK)
    def _gather_chunk(j):
        off = _stage_idx(j)
        poff = off // 8
        for a0_hbm, a1_hbm, ag0_hbm, ag1_hbm in (
                (a0a_hbm, a1a_hbm, ag0a_hbm, ag1a_hbm),
                (a0b_hbm, a1b_hbm, ag0b_hbm, ag1b_hbm)):
            @pl.when(c == 0)
            def _g0():
                pltpu.async_copy(a0_hbm.at[idx_v], rows_v, sem).wait()

            @pl.when(c == 1)
            def _g1():
                pltpu.async_copy(a1_hbm.at[idx_v], rows_v, sem).wait()

            _repack_rows(rows_v, pk_v, _P3_CHUNK)

            @pl.when(c == 0)
            def _st0():
                pltpu.sync_copy(pk_v, ag0_hbm.at[pl.ds(poff, _P3_CHUNK // 8)])

            @pl.when(c == 1)
            def _st1():
                pltpu.sync_copy(pk_v, ag1_hbm.at[pl.ds(poff, _P3_CHUNK // 8)])


def _p3(senders, v0a, v0b, v1a, v1b, az):
    mesh = plsc.VectorSubcoreMesh(core_axis_name="c", subcore_axis_name="s")
    a_t = jax.ShapeDtypeStruct((N, 16), jnp.float32)
    ag_t = jax.ShapeDtypeStruct((_EPK, 128), jnp.float32)
    kfn = pl.kernel(
        _p3_body,
        mesh=mesh,
        out_type=[a_t, a_t, a_t, a_t, ag_t, ag_t, ag_t, ag_t],
        scratch_types=[
            pltpu.VMEM((_P3_CHUNK,), jnp.int32),
            pltpu.VMEM((_P3_CHUNK, 16), jnp.float32),
            pltpu.VMEM((_P3_CHUNK // 8, 128), jnp.float32),
            pltpu.SemaphoreType.DMA,
            pltpu.VMEM_SHARED((N, 16), jnp.float32),
            pltpu.VMEM_SHARED((N, 16), jnp.float32),
        ],
        compiler_params=_SC_PARAMS,
    )
    return kfn(senders, v0a, v0b, v1a, v1b, az)


def _p4_body(ps_ref, pr_ref, ag0a_ref, ag0b_ref, ag1a_ref, ag1b_ref,
             w0t_ref, b0_ref, w1t_ref, b1_ref, woutt_ref, out_ref):
    psT = _unpack16(ps_ref[...])
    prT = _unpack16(pr_ref[...])
    vhat, env, emb16 = _edge_geom(psT, prT)
    z = jnp.dot(w0t_ref[...], emb16, precision=lax.Precision.HIGHEST) + b0_ref[...]
    h = (z * jax.nn.sigmoid(z)) * env
    ag0a = _unpack16(ag0a_ref[...])
    ag0b = _unpack16(ag0b_ref[...])
    ag1a = _unpack16(ag1a_ref[...])
    ag1b = _unpack16(ag1b_ref[...])
    s_lo = (ag0a[0:8] * vhat[0:1] + ag0a[8:16] * vhat[1:2]
            + ag0b[0:8] * vhat[2:3])
    s_hi = (ag1a[0:8] * vhat[0:1] + ag1a[8:16] * vhat[1:2]
            + ag1b[0:8] * vhat[2:3])
    cat = jnp.concatenate([h, s_lo, s_hi], axis=0)
    z2 = jnp.dot(w1t_ref[...], cat, precision=lax.Precision.HIGHEST) + b1_ref[...]
    h2 = (z2 * jax.nn.sigmoid(z2)) * env
    out = jnp.dot(woutt_ref[...], h2, precision=lax.Precision.HIGHEST)
    out_ref[...] = _pack_out(out)


def _p4(ps, pr, ag0a, ag0b, ag1a, ag1b, w0t, b0c, w1t, b1c, woutt):
    grid = (E // _BE,)
    vspec = pl.BlockSpec((_BE // 8, 128), lambda i: (i, 0))
    return pl.pallas_call(
        _p4_body,
        grid=grid,
        in_specs=[
            vspec, vspec, vspec, vspec, vspec, vspec,
            pl.BlockSpec((HID, 16), lambda i: (0, 0)),
            pl.BlockSpec((HID, 1), lambda i: (0, 0)),
            pl.BlockSpec((HID, HID + VHID), lambda i: (0, 0)),
            pl.BlockSpec((HID, 1), lambda i: (0, 0)),
            pl.BlockSpec((1, HID), lambda i: (0, 0)),
        ],
        out_specs=pl.BlockSpec((_BE // 8, 8), lambda i: (i, 0)),
        out_shape=jax.ShapeDtypeStruct((E // 8, 8), jnp.float32),
    )(ps, pr, ag0a, ag0b, ag1a, ag1b, w0t, b0c, w1t, b1c, woutt)


def kernel(species, positions, senders, receivers, W0, b0, Wv, W1, b1, Wout):
    attrs = jax.nn.one_hot(species, 2, dtype=jnp.float32)
    table = jnp.concatenate(
        [positions, attrs, jnp.zeros((N, 11), jnp.float32)], axis=1)
    senders = senders.astype(jnp.int32)
    receivers = receivers.astype(jnp.int32)
    w0t = jnp.concatenate([W0, jnp.zeros((4, HID), jnp.float32)], axis=0).T
    wvt = Wv.T
    w1t = W1.T
    woutt = Wout.T
    b0c = b0.reshape(HID, 1)
    b1c = b1.reshape(HID, 1)
    az = jnp.zeros((N, 16), jnp.float32)

    ps, pr = _sc_gather(table, senders, receivers)
    v0a, v0b, v1a, v1b = _p2(ps, pr, w0t, b0c, wvt)
    (_a0a, _a0b, _a1a, _a1b,
     ag0a, ag0b, ag1a, ag1b) = _p3(senders, v0a, v0b, v1a, v1b, az)
    out_pk = _p4(ps, pr, ag0a, ag0b, ag1a, ag1b,
                 w0t, b0c, w1t, b1c, woutt)
    return out_pk.reshape(E, 1)


# speedup vs baseline: 61.6815x; 1.3189x over previous
import functools
import math

import jax
import jax.numpy as jnp
from jax import lax
from jax.experimental import pallas as pl
from jax.experimental.pallas import tpu as pltpu
from jax.experimental.pallas import tpu_sc as plsc

N = 50000
E = 800000
HID = 64
VHID = 16
NB = 8
RC = 2.0
AVG_NEIGH = 3.0

NC = 2
NS = 16
NW = NC * NS

_SC_PARAMS = pltpu.CompilerParams(use_tc_tiling_on_sc=False)

_EPK = E * 16 // 128


_P1_CHUNK = 1000
_P1_PER_W = E // NW


def _repack_rows(rows_v, pk_v, n_rows):
    @pl.loop(0, n_rows // 8)
    def _row(r):
        for k in range(8):
            pk_v[r, pl.ds(k * 16, 16)] = rows_v[r * 8 + k, :]


def _unpack_rows(pk_v, rows_v, n_rows):
    @pl.loop(0, n_rows // 8)
    def _row(r):
        for k in range(8):
            rows_v[r * 8 + k, :] = pk_v[r, pl.ds(k * 16, 16)]


def _sc_gather_body(table_hbm, senders_hbm, receivers_hbm, ps_hbm, pr_hbm,
                    idx_v, rows_v, pk_v, sem):
    c = lax.axis_index("c")
    s = lax.axis_index("s")
    w = s * NC + c
    base = w * _P1_PER_W
    for src_hbm, dst_hbm in ((senders_hbm, ps_hbm), (receivers_hbm, pr_hbm)):
        @pl.loop(0, _P1_PER_W // _P1_CHUNK)
        def _chunk(j):
            off = base + j * _P1_CHUNK
            pltpu.sync_copy(src_hbm.at[pl.ds(off, _P1_CHUNK)], idx_v)
            pltpu.async_copy(table_hbm.at[idx_v], rows_v, sem).wait()
            _repack_rows(rows_v, pk_v, _P1_CHUNK)
            pltpu.sync_copy(
                pk_v, dst_hbm.at[pl.ds(off // 8, _P1_CHUNK // 8)])


def _sc_gather(table, senders, receivers):
    mesh = plsc.VectorSubcoreMesh(core_axis_name="c", subcore_axis_name="s")
    kfn = pl.kernel(
        _sc_gather_body,
        mesh=mesh,
        out_type=[
            jax.ShapeDtypeStruct((_EPK, 128), jnp.float32),
            jax.ShapeDtypeStruct((_EPK, 128), jnp.float32),
        ],
        scratch_types=[
            pltpu.VMEM((_P1_CHUNK,), jnp.int32),
            pltpu.VMEM((_P1_CHUNK, 16), jnp.float32),
            pltpu.VMEM((_P1_CHUNK // 8, 128), jnp.float32),
            pltpu.SemaphoreType.DMA,
        ],
        compiler_params=_SC_PARAMS,
    )
    return kfn(table, senders, receivers)


_BE = 6400


def _unpack16(pk):
    t = pk.T
    parts = [t[16 * j:16 * j + 16, :] for j in range(8)]
    return jnp.concatenate(parts, axis=1)


def _pack16(x):
    r = x.shape[1] // 8
    parts = [x[:, j * r:(j + 1) * r] for j in range(8)]
    return jnp.concatenate(parts, axis=0).T


def _pack_out(x):
    r = x.shape[1] // 8
    parts = [x[:, j * r:(j + 1) * r] for j in range(8)]
    return jnp.concatenate(parts, axis=0).T


def _edge_geom(psT, prT):
    B = psT.shape[1]
    vec = prT[0:3] - psT[0:3]
    r2 = jnp.sum(vec * vec, axis=0, keepdims=True) + 1e-12
    r = jnp.sqrt(r2)
    rinv = 1.0 / r
    vhat = vec * rinv
    d = r * (1.0 / RC)
    p = 6.0
    env = (1.0
           - ((p + 1.0) * (p + 2.0) / 2.0) * d ** 6
           + p * (p + 2.0) * d ** 7
           - (p * (p + 1.0) / 2.0) * d ** 8)
    env = jnp.where(d < 1.0, env, 0.0)
    x = jnp.pi * d
    s1 = jnp.sin(x)
    two_c = 2.0 * jnp.cos(x)
    sins = [s1, two_c * s1]
    for _ in range(NB - 2):
        sins.append(two_c * sins[-1] - sins[-2])
    bessel = jnp.concatenate(sins, axis=0) * (math.sqrt(2.0 / RC) * rinv)
    emb16 = jnp.concatenate(
        [bessel, psT[3:5], prT[3:5], jnp.zeros((4, B), jnp.float32)], axis=0)
    return vhat, env, emb16


def _p2_body(ps_ref, pr_ref, w0t_ref, b0_ref, wvt_ref,
             v0a_ref, v0b_ref, v1a_ref, v1b_ref):
    psT = _unpack16(ps_ref[...])
    prT = _unpack16(pr_ref[...])
    vhat, env, emb16 = _edge_geom(psT, prT)
    z = jnp.dot(w0t_ref[...], emb16, precision=lax.Precision.HIGHEST) + b0_ref[...]
    h = z * jax.nn.sigmoid(z)
    scale = env * (1.0 / math.sqrt(AVG_NEIGH))
    hwv = jnp.dot(wvt_ref[...], h, precision=lax.Precision.HIGHEST) * scale
    lo = hwv[0:8]
    hi = hwv[8:16]
    zpad = jnp.zeros((8, hwv.shape[1]), jnp.float32)
    v0a_ref[...] = _pack16(
        jnp.concatenate([lo * vhat[0:1], lo * vhat[1:2]], axis=0))
    v0b_ref[...] = _pack16(jnp.concatenate([lo * vhat[2:3], zpad], axis=0))
    v1a_ref[...] = _pack16(
        jnp.concatenate([hi * vhat[0:1], hi * vhat[1:2]], axis=0))
    v1b_ref[...] = _pack16(jnp.concatenate([hi * vhat[2:3], zpad], axis=0))


def _p2(ps, pr, w0t, b0c, wvt):
    grid = (E // _BE,)
    vspec = pl.BlockSpec((_BE // 8, 128), lambda i: (i, 0))
    vshape = jax.ShapeDtypeStruct((_EPK, 128), jnp.float32)
    return pl.pallas_call(
        _p2_body,
        grid=grid,
        in_specs=[
            vspec,
            vspec,
            pl.BlockSpec((HID, 16), lambda i: (0, 0)),
            pl.BlockSpec((HID, 1), lambda i: (0, 0)),
            pl.BlockSpec((VHID, HID), lambda i: (0, 0)),
        ],
        out_specs=[vspec, vspec, vspec, vspec],
        out_shape=[vshape, vshape, vshape, vshape],
    )(ps, pr, w0t, b0c, wvt)


_P3_CHUNK = 1000
_P3_PER_T = E // NS
_P3_NCH = _P3_PER_T // _P3_CHUNK


def _p3_body(senders_hbm, v0a_hbm, v0b_hbm, v1a_hbm, v1b_hbm, az_hbm,
             a0a_hbm, a0b_hbm, a1a_hbm, a1b_hbm,
             ag0a_hbm, ag0b_hbm, ag1a_hbm, ag1b_hbm,
             idx0, idx1, rows_a, rows_b, pk_a, pk_b,
             semi0, semi1, sema, semb, a_sh):
    c = lax.axis_index("c")
    s = lax.axis_index("s")
    base = s * _P3_PER_T
    idxs = (idx0, idx1)
    semis = (semi0, semi1)
    pks = (pk_a, pk_b)
    rows = (rows_a, rows_b)
    sems = (sema, semb)

    def start_idx(ch, slot):
        @pl.when(ch < _P3_NCH)
        def _():
            pltpu.async_copy(
                senders_hbm.at[pl.ds(base + ch * _P3_CHUNK, _P3_CHUNK)],
                idxs[slot], semis[slot])

    def wait_idx(slot):
        pltpu.make_async_copy(
            senders_hbm.at[pl.ds(0, _P3_CHUNK)], idxs[slot],
            semis[slot]).wait()

    for vc0, vc1, ac0, ac1 in ((v0a_hbm, v1a_hbm, a0a_hbm, a1a_hbm),
                               (v0b_hbm, v1b_hbm, a0b_hbm, a1b_hbm)):
        @pl.when(s == 0)
        def _zero():
            pltpu.sync_copy(az_hbm, a_sh)

        plsc.subcore_barrier()

        def start_pk(ch, slot, vc0=vc0, vc1=vc1):
            @pl.when(ch < _P3_NCH)
            def _():
                pslice = pl.ds((base + ch * _P3_CHUNK) // 8, _P3_CHUNK // 8)

                @pl.when(c == 0)
                def _l0():
                    pltpu.async_copy(vc0.at[pslice], pks[slot], sems[slot])

                @pl.when(c == 1)
                def _l1():
                    pltpu.async_copy(vc1.at[pslice], pks[slot], sems[slot])

        def wait_pk(slot, vc0=vc0):
            pltpu.make_async_copy(
                vc0.at[pl.ds(0, _P3_CHUNK // 8)], pks[slot],
                sems[slot]).wait()

        start_idx(0, 0)
        start_pk(0, 0)
        start_idx(1, 1)
        start_pk(1, 1)

        @pl.loop(0, _P3_NCH // 2)
        def _pair(g):
            for slot in (0, 1):
                ch = 2 * g + slot
                wait_idx(slot)
                wait_pk(slot)
                _unpack_rows(pks[slot], rows[slot], _P3_CHUNK)
                pltpu.sync_copy(rows[slot], a_sh.at[idxs[slot]], add=True)
                start_idx(ch + 2, slot)
                start_pk(ch + 2, slot)

        plsc.subcore_barrier()

        @pl.when(s == 0)
        def _writeback():
            @pl.when(c == 0)
            def _w0():
                pltpu.sync_copy(a_sh, ac0)

            @pl.when(c == 1)
            def _w1():
                pltpu.sync_copy(a_sh, ac1)

        plsc.subcore_barrier()

    start_idx(0, 0)
    start_idx(1, 1)

    @pl.loop(0, _P3_NCH // 2)
    def _gpair(g):
        for slot in (0, 1):
            ch = 2 * g + slot
            poff = (base + ch * _P3_CHUNK) // 8
            wait_idx(slot)

            @pl.when(c == 0)
            def _ga0():
                pltpu.async_copy(a0a_hbm.at[idxs[slot]], rows_a, sema)
                pltpu.async_copy(a0b_hbm.at[idxs[slot]], rows_b, semb)

            @pl.when(c == 1)
            def _ga1():
                pltpu.async_copy(a1a_hbm.at[idxs[slot]], rows_a, sema)
                pltpu.async_copy(a1b_hbm.at[idxs[slot]], rows_b, semb)

            pltpu.make_async_copy(
                a0a_hbm.at[pl.ds(0, _P3_CHUNK)], rows_a, sema).wait()
            _repack_rows(rows_a, pk_a, _P3_CHUNK)

            @pl.when(c == 0)
            def _sa0():
                pltpu.sync_copy(pk_a, ag0a_hbm.at[pl.ds(poff, _P3_CHUNK // 8)])

            @pl.when(c == 1)
            def _sa1():
                pltpu.sync_copy(pk_a, ag1a_hbm.at[pl.ds(poff, _P3_CHUNK // 8)])

            pltpu.make_async_copy(
                a0b_hbm.at[pl.ds(0, _P3_CHUNK)], rows_b, semb).wait()
            start_idx(ch + 2, slot)
            _repack_rows(rows_b, pk_b, _P3_CHUNK)

            @pl.when(c == 0)
            def _sb0():
                pltpu.sync_copy(pk_b, ag0b_hbm.at[pl.ds(poff, _P3_CHUNK // 8)])

            @pl.when(c == 1)
            def _sb1():
                pltpu.sync_copy(pk_b, ag1b_hbm.at[pl.ds(poff, _P3_CHUNK // 8)])


def _p3(senders, v0a, v0b, v1a, v1b, az):
    mesh = plsc.VectorSubcoreMesh(core_axis_name="c", subcore_axis_name="s")
    a_t = jax.ShapeDtypeStruct((N, 16), jnp.float32)
    ag_t = jax.ShapeDtypeStruct((_EPK, 128), jnp.float32)
    kfn = pl.kernel(
        _p3_body,
        mesh=mesh,
        out_type=[a_t, a_t, a_t, a_t, ag_t, ag_t, ag_t, ag_t],
        scratch_types=[
            pltpu.VMEM((_P3_CHUNK,), jnp.int32),
            pltpu.VMEM((_P3_CHUNK,), jnp.int32),
            pltpu.VMEM((_P3_CHUNK, 16), jnp.float32),
            pltpu.VMEM((_P3_CHUNK, 16), jnp.float32),
            pltpu.VMEM((_P3_CHUNK // 8, 128), jnp.float32),
            pltpu.VMEM((_P3_CHUNK // 8, 128), jnp.float32),
            pltpu.SemaphoreType.DMA,
            pltpu.SemaphoreType.DMA,
            pltpu.SemaphoreType.DMA,
            pltpu.SemaphoreType.DMA,
            pltpu.VMEM_SHARED((N, 16), jnp.float32),
        ],
        compiler_params=_SC_PARAMS,
    )
    return kfn(senders, v0a, v0b, v1a, v1b, az)


def _p4_body(ps_ref, pr_ref, ag0a_ref, ag0b_ref, ag1a_ref, ag1b_ref,
             w0t_ref, b0_ref, w1t_ref, b1_ref, woutt_ref, out_ref):
    psT = _unpack16(ps_ref[...])
    prT = _unpack16(pr_ref[...])
    vhat, env, emb16 = _edge_geom(psT, prT)
    z = jnp.dot(w0t_ref[...], emb16, precision=lax.Precision.HIGHEST) + b0_ref[...]
    h = (z * jax.nn.sigmoid(z)) * env
    ag0a = _unpack16(ag0a_ref[...])
    ag0b = _unpack16(ag0b_ref[...])
    ag1a = _unpack16(ag1a_ref[...])
    ag1b = _unpack16(ag1b_ref[...])
    s_lo = (ag0a[0:8] * vhat[0:1] + ag0a[8:16] * vhat[1:2]
            + ag0b[0:8] * vhat[2:3])
    s_hi = (ag1a[0:8] * vhat[0:1] + ag1a[8:16] * vhat[1:2]
            + ag1b[0:8] * vhat[2:3])
    cat = jnp.concatenate([h, s_lo, s_hi], axis=0)
    z2 = jnp.dot(w1t_ref[...], cat, precision=lax.Precision.HIGHEST) + b1_ref[...]
    h2 = (z2 * jax.nn.sigmoid(z2)) * env
    out = jnp.dot(woutt_ref[...], h2, precision=lax.Precision.HIGHEST)
    out_ref[...] = _pack_out(out)


def _p4(ps, pr, ag0a, ag0b, ag1a, ag1b, w0t, b0c, w1t, b1c, woutt):
    grid = (E // _BE,)
    vspec = pl.BlockSpec((_BE // 8, 128), lambda i: (i, 0))
    return pl.pallas_call(
        _p4_body,
        grid=grid,
        in_specs=[
            vspec, vspec, vspec, vspec, vspec, vspec,
            pl.BlockSpec((HID, 16), lambda i: (0, 0)),
            pl.BlockSpec((HID, 1), lambda i: (0, 0)),
            pl.BlockSpec((HID, HID + VHID), lambda i: (0, 0)),
            pl.BlockSpec((HID, 1), lambda i: (0, 0)),
            pl.BlockSpec((1, HID), lambda i: (0, 0)),
        ],
        out_specs=pl.BlockSpec((_BE // 8, 8), lambda i: (i, 0)),
        out_shape=jax.ShapeDtypeStruct((E // 8, 8), jnp.float32),
    )(ps, pr, ag0a, ag0b, ag1a, ag1b, w0t, b0c, w1t, b1c, woutt)


def kernel(species, positions, senders, receivers, W0, b0, Wv, W1, b1, Wout):
    attrs = jax.nn.one_hot(species, 2, dtype=jnp.float32)
    table = jnp.concatenate(
        [positions, attrs, jnp.zeros((N, 11), jnp.float32)], axis=1)
    senders = senders.astype(jnp.int32)
    receivers = receivers.astype(jnp.int32)
    w0t = jnp.concatenate([W0, jnp.zeros((4, HID), jnp.float32)], axis=0).T
    wvt = Wv.T
    w1t = W1.T
    woutt = Wout.T
    b0c = b0.reshape(HID, 1)
    b1c = b1.reshape(HID, 1)
    az = jnp.zeros((N, 16), jnp.float32)

    ps, pr = _sc_gather(table, senders, receivers)
    v0a, v0b, v1a, v1b = _p2(ps, pr, w0t, b0c, wvt)
    (_a0a, _a0b, _a1a, _a1b,
     ag0a, ag0b, ag1a, ag1b) = _p3(senders, v0a, v0b, v1a, v1b, az)
    out_pk = _p4(ps, pr, ag0a, ag0b, ag1a, ag1b,
                 w0t, b0c, w1t, b1c, woutt)
    return out_pk.reshape(E, 1)


# speedup vs baseline: 63.7188x; 1.0330x over previous
import functools
import math

import jax
import jax.numpy as jnp
from jax import lax
from jax.experimental import pallas as pl
from jax.experimental.pallas import tpu as pltpu
from jax.experimental.pallas import tpu_sc as plsc

N = 50000
E = 800000
HID = 64
VHID = 16
NB = 8
RC = 2.0
AVG_NEIGH = 3.0

NC = 2
NS = 16
NW = NC * NS

_SC_PARAMS = pltpu.CompilerParams(use_tc_tiling_on_sc=False)

_EPK = E * 16 // 128


_P1_CHUNK = 1000
_P1_PER_W = E // NW


def _repack_rows(rows_v, pk_v, n_rows):
    @pl.loop(0, n_rows // 8)
    def _row(r):
        for k in range(8):
            pk_v[r, pl.ds(k * 16, 16)] = rows_v[r * 8 + k, :]


def _unpack_rows(pk_v, rows_v, n_rows):
    @pl.loop(0, n_rows // 8)
    def _row(r):
        for k in range(8):
            rows_v[r * 8 + k, :] = pk_v[r, pl.ds(k * 16, 16)]


def _sc_gather_body(table_hbm, senders_hbm, receivers_hbm, ps_hbm, pr_hbm,
                    idx_v, rows_v, pk_v, sem):
    c = lax.axis_index("c")
    s = lax.axis_index("s")
    w = s * NC + c
    base = w * _P1_PER_W
    for src_hbm, dst_hbm in ((senders_hbm, ps_hbm), (receivers_hbm, pr_hbm)):
        @pl.loop(0, _P1_PER_W // _P1_CHUNK)
        def _chunk(j):
            off = base + j * _P1_CHUNK
            pltpu.sync_copy(src_hbm.at[pl.ds(off, _P1_CHUNK)], idx_v)
            pltpu.async_copy(table_hbm.at[idx_v], rows_v, sem).wait()
            _repack_rows(rows_v, pk_v, _P1_CHUNK)
            pltpu.sync_copy(
                pk_v, dst_hbm.at[pl.ds(off // 8, _P1_CHUNK // 8)])


def _sc_gather(table, senders, receivers):
    mesh = plsc.VectorSubcoreMesh(core_axis_name="c", subcore_axis_name="s")
    kfn = pl.kernel(
        _sc_gather_body,
        mesh=mesh,
        out_type=[
            jax.ShapeDtypeStruct((_EPK, 128), jnp.float32),
            jax.ShapeDtypeStruct((_EPK, 128), jnp.float32),
        ],
        scratch_types=[
            pltpu.VMEM((_P1_CHUNK,), jnp.int32),
            pltpu.VMEM((_P1_CHUNK, 16), jnp.float32),
            pltpu.VMEM((_P1_CHUNK // 8, 128), jnp.float32),
            pltpu.SemaphoreType.DMA,
        ],
        compiler_params=_SC_PARAMS,
    )
    return kfn(table, senders, receivers)


_BE = 6400


def _unpack16(pk):
    t = pk.T
    parts = [t[16 * j:16 * j + 16, :] for j in range(8)]
    return jnp.concatenate(parts, axis=1)


def _pack16(x):
    r = x.shape[1] // 8
    parts = [x[:, j * r:(j + 1) * r] for j in range(8)]
    return jnp.concatenate(parts, axis=0).T


def _pack_out(x):
    r = x.shape[1] // 8
    parts = [x[:, j * r:(j + 1) * r] for j in range(8)]
    return jnp.concatenate(parts, axis=0).T


def _edge_geom(psT, prT):
    B = psT.shape[1]
    vec = prT[0:3] - psT[0:3]
    r2 = jnp.sum(vec * vec, axis=0, keepdims=True) + 1e-12
    r = jnp.sqrt(r2)
    rinv = 1.0 / r
    vhat = vec * rinv
    d = r * (1.0 / RC)
    p = 6.0
    env = (1.0
           - ((p + 1.0) * (p + 2.0) / 2.0) * d ** 6
           + p * (p + 2.0) * d ** 7
           - (p * (p + 1.0) / 2.0) * d ** 8)
    env = jnp.where(d < 1.0, env, 0.0)
    x = jnp.minimum(jnp.pi * d, jnp.pi)
    u = x * x
    _sc = [1.0 / math.factorial(k) for k in range(18)]
    s1 = x * (1.0 + u * (-_sc[3] + u * (_sc[5] + u * (-_sc[7] + u * (
        _sc[9] + u * (-_sc[11] + u * (_sc[13] - u * _sc[15])))))))
    two_c = 2.0 * (1.0 + u * (-_sc[2] + u * (_sc[4] + u * (-_sc[6] + u * (
        _sc[8] + u * (-_sc[10] + u * (_sc[12] + u * (-_sc[14] + u * _sc[16]))))))))
    sins = [s1, two_c * s1]
    for _ in range(NB - 2):
        sins.append(two_c * sins[-1] - sins[-2])
    bessel = jnp.concatenate(sins, axis=0) * (math.sqrt(2.0 / RC) * rinv)
    emb16 = jnp.concatenate(
        [bessel, psT[3:5], prT[3:5], jnp.zeros((4, B), jnp.float32)], axis=0)
    return vhat, env, emb16


def _p2_body(ps_ref, pr_ref, w0t_ref, b0_ref, wvt_ref,
             v0a_ref, v0b_ref, v1a_ref, v1b_ref):
    psT = _unpack16(ps_ref[...])
    prT = _unpack16(pr_ref[...])
    vhat, env, emb16 = _edge_geom(psT, prT)
    z = jnp.dot(w0t_ref[...], emb16, precision=lax.Precision.HIGHEST) + b0_ref[...]
    h = z * jax.nn.sigmoid(z)
    scale = env * (1.0 / math.sqrt(AVG_NEIGH))
    hwv = jnp.dot(wvt_ref[...], h, precision=lax.Precision.HIGHEST) * scale
    lo = hwv[0:8]
    hi = hwv[8:16]
    zpad = jnp.zeros((8, hwv.shape[1]), jnp.float32)
    v0a_ref[...] = _pack16(
        jnp.concatenate([lo * vhat[0:1], lo * vhat[1:2]], axis=0))
    v0b_ref[...] = _pack16(jnp.concatenate([lo * vhat[2:3], zpad], axis=0))
    v1a_ref[...] = _pack16(
        jnp.concatenate([hi * vhat[0:1], hi * vhat[1:2]], axis=0))
    v1b_ref[...] = _pack16(jnp.concatenate([hi * vhat[2:3], zpad], axis=0))


def _p2(ps, pr, w0t, b0c, wvt):
    grid = (E // _BE,)
    vspec = pl.BlockSpec((_BE // 8, 128), lambda i: (i, 0))
    vshape = jax.ShapeDtypeStruct((_EPK, 128), jnp.float32)
    return pl.pallas_call(
        _p2_body,
        grid=grid,
        in_specs=[
            vspec,
            vspec,
            pl.BlockSpec((HID, 16), lambda i: (0, 0)),
            pl.BlockSpec((HID, 1), lambda i: (0, 0)),
            pl.BlockSpec((VHID, HID), lambda i: (0, 0)),
        ],
        out_specs=[vspec, vspec, vspec, vspec],
        out_shape=[vshape, vshape, vshape, vshape],
    )(ps, pr, w0t, b0c, wvt)


_P3_CHUNK = 1000
_P3_PER_T = E // NS
_P3_NCH = _P3_PER_T // _P3_CHUNK


def _p3_body(senders_hbm, v0a_hbm, v0b_hbm, v1a_hbm, v1b_hbm, az_hbm,
             a0a_hbm, a0b_hbm, a1a_hbm, a1b_hbm,
             ag0a_hbm, ag0b_hbm, ag1a_hbm, ag1b_hbm,
             idx0, idx1, rows_a, rows_b, pk_a, pk_b,
             semi0, semi1, sema, semb, a_sh):
    c = lax.axis_index("c")
    s = lax.axis_index("s")
    base = s * _P3_PER_T
    idxs = (idx0, idx1)
    semis = (semi0, semi1)
    pks = (pk_a, pk_b)
    rows = (rows_a, rows_b)
    sems = (sema, semb)

    def start_idx(ch, slot):
        @pl.when(ch < _P3_NCH)
        def _():
            pltpu.async_copy(
                senders_hbm.at[pl.ds(base + ch * _P3_CHUNK, _P3_CHUNK)],
                idxs[slot], semis[slot])

    def wait_idx(slot):
        pltpu.make_async_copy(
            senders_hbm.at[pl.ds(0, _P3_CHUNK)], idxs[slot],
            semis[slot]).wait()

    for vc0, vc1, ac0, ac1 in ((v0a_hbm, v1a_hbm, a0a_hbm, a1a_hbm),
                               (v0b_hbm, v1b_hbm, a0b_hbm, a1b_hbm)):
        @pl.when(s == 0)
        def _zero():
            pltpu.sync_copy(az_hbm, a_sh)

        plsc.subcore_barrier()

        def start_pk(ch, slot, vc0=vc0, vc1=vc1):
            @pl.when(ch < _P3_NCH)
            def _():
                pslice = pl.ds((base + ch * _P3_CHUNK) // 8, _P3_CHUNK // 8)

                @pl.when(c == 0)
                def _l0():
                    pltpu.async_copy(vc0.at[pslice], pks[slot], sems[slot])

                @pl.when(c == 1)
                def _l1():
                    pltpu.async_copy(vc1.at[pslice], pks[slot], sems[slot])

        def wait_pk(slot, vc0=vc0):
            pltpu.make_async_copy(
                vc0.at[pl.ds(0, _P3_CHUNK // 8)], pks[slot],
                sems[slot]).wait()

        start_idx(0, 0)
        start_pk(0, 0)
        start_idx(1, 1)
        start_pk(1, 1)

        @pl.loop(0, _P3_NCH // 2)
        def _pair(g):
            for slot in (0, 1):
                ch = 2 * g + slot
                wait_idx(slot)
                wait_pk(slot)
                _unpack_rows(pks[slot], rows[slot], _P3_CHUNK)
                pltpu.sync_copy(rows[slot], a_sh.at[idxs[slot]], add=True)
                start_idx(ch + 2, slot)
                start_pk(ch + 2, slot)

        plsc.subcore_barrier()

        @pl.when(s == 0)
        def _writeback():
            @pl.when(c == 0)
            def _w0():
                pltpu.sync_copy(a_sh, ac0)

            @pl.when(c == 1)
            def _w1():
                pltpu.sync_copy(a_sh, ac1)

        plsc.subcore_barrier()

    start_idx(0, 0)
    start_idx(1, 1)

    @pl.loop(0, _P3_NCH // 2)
    def _gpair(g):
        for slot in (0, 1):
            ch = 2 * g + slot
            poff = (base + ch * _P3_CHUNK) // 8
            wait_idx(slot)

            @pl.when(c == 0)
            def _ga0():
                pltpu.async_copy(a0a_hbm.at[idxs[slot]], rows_a, sema)
                pltpu.async_copy(a0b_hbm.at[idxs[slot]], rows_b, semb)

            @pl.when(c == 1)
            def _ga1():
                pltpu.async_copy(a1a_hbm.at[idxs[slot]], rows_a, sema)
                pltpu.async_copy(a1b_hbm.at[idxs[slot]], rows_b, semb)

            pltpu.make_async_copy(
                a0a_hbm.at[pl.ds(0, _P3_CHUNK)], rows_a, sema).wait()
            _repack_rows(rows_a, pk_a, _P3_CHUNK)

            @pl.when(c == 0)
            def _sa0():
                pltpu.sync_copy(pk_a, ag0a_hbm.at[pl.ds(poff, _P3_CHUNK // 8)])

            @pl.when(c == 1)
            def _sa1():
                pltpu.sync_copy(pk_a, ag1a_hbm.at[pl.ds(poff, _P3_CHUNK // 8)])

            pltpu.make_async_copy(
                a0b_hbm.at[pl.ds(0, _P3_CHUNK)], rows_b, semb).wait()
            start_idx(ch + 2, slot)
            _repack_rows(rows_b, pk_b, _P3_CHUNK)

            @pl.when(c == 0)
            def _sb0():
                pltpu.sync_copy(pk_b, ag0b_hbm.at[pl.ds(poff, _P3_CHUNK // 8)])

            @pl.when(c == 1)
            def _sb1():
                pltpu.sync_copy(pk_b, ag1b_hbm.at[pl.ds(poff, _P3_CHUNK // 8)])


def _p3(senders, v0a, v0b, v1a, v1b, az):
    mesh = plsc.VectorSubcoreMesh(core_axis_name="c", subcore_axis_name="s")
    a_t = jax.ShapeDtypeStruct((N, 16), jnp.float32)
    ag_t = jax.ShapeDtypeStruct((_EPK, 128), jnp.float32)
    kfn = pl.kernel(
        _p3_body,
        mesh=mesh,
        out_type=[a_t, a_t, a_t, a_t, ag_t, ag_t, ag_t, ag_t],
        scratch_types=[
            pltpu.VMEM((_P3_CHUNK,), jnp.int32),
            pltpu.VMEM((_P3_CHUNK,), jnp.int32),
            pltpu.VMEM((_P3_CHUNK, 16), jnp.float32),
            pltpu.VMEM((_P3_CHUNK, 16), jnp.float32),
            pltpu.VMEM((_P3_CHUNK // 8, 128), jnp.float32),
            pltpu.VMEM((_P3_CHUNK // 8, 128), jnp.float32),
            pltpu.SemaphoreType.DMA,
            pltpu.SemaphoreType.DMA,
            pltpu.SemaphoreType.DMA,
            pltpu.SemaphoreType.DMA,
            pltpu.VMEM_SHARED((N, 16), jnp.float32),
        ],
        compiler_params=_SC_PARAMS,
    )
    return kfn(senders, v0a, v0b, v1a, v1b, az)


def _p4_body(ps_ref, pr_ref, ag0a_ref, ag0b_ref, ag1a_ref, ag1b_ref,
             w0t_ref, b0_ref, w1t_ref, b1_ref, woutt_ref, out_ref):
    psT = _unpack16(ps_ref[...])
    prT = _unpack16(pr_ref[...])
    vhat, env, emb16 = _edge_geom(psT, prT)
    z = jnp.dot(w0t_ref[...], emb16, precision=lax.Precision.HIGHEST) + b0_ref[...]
    h = (z * jax.nn.sigmoid(z)) * env
    ag0a = _unpack16(ag0a_ref[...])
    ag0b = _unpack16(ag0b_ref[...])
    ag1a = _unpack16(ag1a_ref[...])
    ag1b = _unpack16(ag1b_ref[...])
    s_lo = (ag0a[0:8] * vhat[0:1] + ag0a[8:16] * vhat[1:2]
            + ag0b[0:8] * vhat[2:3])
    s_hi = (ag1a[0:8] * vhat[0:1] + ag1a[8:16] * vhat[1:2]
            + ag1b[0:8] * vhat[2:3])
    cat = jnp.concatenate([h, s_lo, s_hi], axis=0)
    z2 = jnp.dot(w1t_ref[...], cat, precision=lax.Precision.HIGHEST) + b1_ref[...]
    h2 = (z2 * jax.nn.sigmoid(z2)) * env
    out = jnp.dot(woutt_ref[...], h2, precision=lax.Precision.HIGHEST)
    out_ref[...] = _pack_out(out)


def _p4(ps, pr, ag0a, ag0b, ag1a, ag1b, w0t, b0c, w1t, b1c, woutt):
    grid = (E // _BE,)
    vspec = pl.BlockSpec((_BE // 8, 128), lambda i: (i, 0))
    return pl.pallas_call(
        _p4_body,
        grid=grid,
        in_specs=[
            vspec, vspec, vspec, vspec, vspec, vspec,
            pl.BlockSpec((HID, 16), lambda i: (0, 0)),
            pl.BlockSpec((HID, 1), lambda i: (0, 0)),
            pl.BlockSpec((HID, HID + VHID), lambda i: (0, 0)),
            pl.BlockSpec((HID, 1), lambda i: (0, 0)),
            pl.BlockSpec((1, HID), lambda i: (0, 0)),
        ],
        out_specs=pl.BlockSpec((_BE // 8, 8), lambda i: (i, 0)),
        out_shape=jax.ShapeDtypeStruct((E // 8, 8), jnp.float32),
    )(ps, pr, ag0a, ag0b, ag1a, ag1b, w0t, b0c, w1t, b1c, woutt)


def kernel(species, positions, senders, receivers, W0, b0, Wv, W1, b1, Wout):
    attrs = jax.nn.one_hot(species, 2, dtype=jnp.float32)
    table = jnp.concatenate(
        [positions, attrs, jnp.zeros((N, 11), jnp.float32)], axis=1)
    senders = senders.astype(jnp.int32)
    receivers = receivers.astype(jnp.int32)
    w0t = jnp.concatenate([W0, jnp.zeros((4, HID), jnp.float32)], axis=0).T
    wvt = Wv.T
    w1t = W1.T
    woutt = Wout.T
    b0c = b0.reshape(HID, 1)
    b1c = b1.reshape(HID, 1)
    az = jnp.zeros((N, 16), jnp.float32)

    ps, pr = _sc_gather(table, senders, receivers)
    v0a, v0b, v1a, v1b = _p2(ps, pr, w0t, b0c, wvt)
    (_a0a, _a0b, _a1a, _a1b,
     ag0a, ag0b, ag1a, ag1b) = _p3(senders, v0a, v0b, v1a, v1b, az)
    out_pk = _p4(ps, pr, ag0a, ag0b, ag1a, ag1b,
                 w0t, b0c, w1t, b1c, woutt)
    return out_pk.reshape(E, 1)
